# Initial kernel scaffold; baseline (speedup 1.0000x reference)
import jax
import jax.numpy as jnp
from jax.experimental import pallas as pl


def kernel(edge_index, adj_vals, x, alpha, W):
    raise NotImplementedError("write your pallas kernel here")


# speedup vs baseline: 3.6858x; 3.6858x over previous
import functools

import jax
import jax.numpy as jnp
from jax import lax
from jax.experimental import pallas as pl
from jax.experimental.pallas import tpu as pltpu
from jax.experimental.pallas import tpu_sc as plsc

NC = 2
NS = 16
L = 16
CHUNK = 128


def _sc_spmm_body(src_hbm, dst_hbm, vals_hbm, x_hbm, zeros_hbm, part_hbm,
                  acc_sh, src_v, dst_v, vals_v, rows_v, sem):
  n, d = x_hbm.shape
  ept = src_hbm.shape[0] // (NC * NS)
  nchunks = ept // CHUNK
  zrows = (n // NS) // 8 * 8
  rem = n - NS * zrows
  c = lax.axis_index("c")
  s = lax.axis_index("s")

  pltpu.sync_copy(zeros_hbm.at[pl.ds(s * zrows, zrows)],
                  acc_sh.at[pl.ds(s * zrows, zrows)])
  if rem:
    @pl.when(s == 0)
    def _():
      pltpu.sync_copy(zeros_hbm.at[pl.ds(NS * zrows, rem)],
                      acc_sh.at[pl.ds(NS * zrows, rem)])
  plsc.subcore_barrier()

  tile_base = (c * NS + s) * ept

  def chunk_body(k, carry):
    base = tile_base + k * CHUNK
    pltpu.sync_copy(src_hbm.at[pl.ds(base, CHUNK)], src_v)
    pltpu.sync_copy(dst_hbm.at[pl.ds(base, CHUNK)], dst_v)
    pltpu.sync_copy(vals_hbm.at[pl.ds(base, CHUNK)], vals_v)
    pltpu.async_copy(x_hbm.at[src_v], rows_v, sem).wait()

    for g in range(CHUNK // L):
      vv = vals_v[pl.ds(g * L, L)]
      for j in range(L):
        e_row = g * L + j
        vj = lax.gather(
            vv, jnp.full((L, 1), j, jnp.int32),
            lax.GatherDimensionNumbers(offset_dims=(),
                                       collapsed_slice_dims=(0,),
                                       start_index_map=(0,)),
            slice_sizes=(1,),
            mode=lax.GatherScatterMode.PROMISE_IN_BOUNDS)
        for k in range(d // L):
          sl = (e_row, pl.ds(k * L, L))
          rows_v[sl] = rows_v[sl] * vj
    pltpu.sync_copy(rows_v, acc_sh.at[dst_v], add=True)
    return carry

  lax.fori_loop(0, nchunks, chunk_body, 0)
  plsc.subcore_barrier()
  pltpu.sync_copy(acc_sh.at[pl.ds(s * zrows, zrows)],
                  part_hbm.at[pl.ds(c * n + s * zrows, zrows)])
  if rem:
    @pl.when(s == 0)
    def _():
      pltpu.sync_copy(acc_sh.at[pl.ds(NS * zrows, rem)],
                      part_hbm.at[pl.ds(c * n + NS * zrows, rem)])


def _tc_finish_body(a_ref, p_ref, x_ref, w_ref, o_ref):
  a = a_ref[0]
  blended = (1.0 - a) * (p_ref[0] + p_ref[1]) + a * x_ref[...]
  o_ref[...] = jnp.dot(blended, w_ref[...], preferred_element_type=jnp.float32)


def kernel(edge_index, adj_vals, x, alpha, W):
  n, d_in = x.shape
  d_out = W.shape[1]
  e = adj_vals.shape[0]

  dst = edge_index[0]
  src = edge_index[1]
  ept = -(-e // (NC * NS * CHUNK)) * CHUNK
  e_pad = ept * NC * NS
  if e_pad != e:
    pad = e_pad - e
    src = jnp.concatenate([src, jnp.zeros((pad,), src.dtype)])
    dst = jnp.concatenate([dst, jnp.zeros((pad,), dst.dtype)])
    vals = jnp.concatenate([adj_vals, jnp.zeros((pad,), adj_vals.dtype)])
  else:
    vals = adj_vals
  zeros = jnp.zeros((n, d_in), jnp.float32)

  mesh = plsc.VectorSubcoreMesh(core_axis_name="c", subcore_axis_name="s")
  part = pl.kernel(
      _sc_spmm_body,
      out_type=jax.ShapeDtypeStruct((NC * n, d_in), jnp.float32),
      mesh=mesh,
      scratch_types=[
          pltpu.VMEM_SHARED((n, d_in), jnp.float32),
          pltpu.VMEM((CHUNK,), jnp.int32),
          pltpu.VMEM((CHUNK,), jnp.int32),
          pltpu.VMEM((CHUNK,), jnp.float32),
          pltpu.VMEM((CHUNK, d_in), jnp.float32),
          pltpu.SemaphoreType.DMA,
      ],
  )(src, dst, vals, x, zeros)

  part = part.reshape(NC, n, d_in)

  bt = 400
  grid = n // bt
  out = pl.pallas_call(
      _tc_finish_body,
      out_shape=jax.ShapeDtypeStruct((n, d_out), jnp.float32),
      grid=(grid,),
      in_specs=[
          pl.BlockSpec(memory_space=pltpu.SMEM),
          pl.BlockSpec((NC, bt, d_in), lambda i: (0, i, 0)),
          pl.BlockSpec((bt, d_in), lambda i: (i, 0)),
          pl.BlockSpec((d_in, d_out), lambda i: (0, 0)),
      ],
      out_specs=pl.BlockSpec((bt, d_out), lambda i: (i, 0)),
  )(alpha.reshape(1), part, x, W)
  return out
